# Initial kernel scaffold; baseline (speedup 1.0000x reference)
import jax
import jax.numpy as jnp
from jax.experimental import pallas as pl


def kernel(features, edge_index, edge_weight, mask, W1, b1, W2, b2, W3, b3):
    raise NotImplementedError("write your pallas kernel here")


# speedup vs baseline: 3.6017x; 3.6017x over previous
import functools

import jax
import jax.numpy as jnp
from jax import lax
from jax.experimental import pallas as pl
from jax.experimental.pallas import tpu as pltpu
from jax.experimental.pallas import tpu_sc as plsc

NNODE = 10000
NP = 10240
D = 128
NE = 320000
NC, NS, LANES = 2, 16, 16
NW = NC * NS
CH = 128
CPT = 79
EP = NW * CPT * CH
ROWS_PT = NP // NS
DUMMY = NNODE

_mesh = plsc.VectorSubcoreMesh(
    core_axis_name="c", subcore_axis_name="s", num_cores=NC, num_subcores=NS)
_sc_params = pltpu.CompilerParams(needs_layout_passes=False)


def _wid(cid, sid):
  return sid * NC + cid


@functools.partial(
    pl.kernel,
    out_type=jax.ShapeDtypeStruct((NC, 4, NP), jnp.float32),
    mesh=_mesh,
    compiler_params=_sc_params,
    scratch_types=[
        pltpu.VMEM((CPT, CH), jnp.int32),
        pltpu.VMEM((CPT, CH), jnp.int32),
        pltpu.VMEM((CPT, CH), jnp.float32),
        pltpu.VMEM((CPT, CH), jnp.float32),
        pltpu.VMEM((ROWS_PT,), jnp.float32),
        pltpu.VMEM_SHARED((NP,), jnp.float32),
        pltpu.VMEM_SHARED((NP,), jnp.float32),
        pltpu.VMEM_SHARED((NP,), jnp.float32),
        pltpu.VMEM_SHARED((NP,), jnp.float32),
    ],
)
def _deg_kernel(src_h, dst_h, ew_h, one_h, out_h,
                srcv, dstv, ewv, onev, zrow, swdeg, dwdeg, dout, din):
  cid = lax.axis_index("c")
  sid = lax.axis_index("s")
  wid = _wid(cid, sid)
  base = sid * ROWS_PT

  for i in range(ROWS_PT // 16):
    zrow[pl.ds(i * 16, 16)] = jnp.zeros((16,), jnp.float32)
  for acc in (swdeg, dwdeg, dout, din):
    pltpu.sync_copy(zrow, acc.at[pl.ds(base, ROWS_PT)])

  pltpu.sync_copy(src_h.at[wid], srcv)
  pltpu.sync_copy(dst_h.at[wid], dstv)
  pltpu.sync_copy(ew_h.at[wid], ewv)
  pltpu.sync_copy(one_h.at[wid], onev)
  plsc.subcore_barrier()

  def chunk(j, carry):
    pltpu.sync_copy(ewv.at[j], swdeg.at[srcv.at[j]], add=True)
    pltpu.sync_copy(ewv.at[j], dwdeg.at[dstv.at[j]], add=True)
    pltpu.sync_copy(onev.at[j], dout.at[srcv.at[j]], add=True)
    pltpu.sync_copy(onev.at[j], din.at[dstv.at[j]], add=True)
    return carry

  lax.fori_loop(0, CPT, chunk, 0)
  plsc.subcore_barrier()

  for row, acc in enumerate((swdeg, dwdeg, dout, din)):
    pltpu.sync_copy(acc.at[pl.ds(base, ROWS_PT)],
                    out_h.at[cid, row, pl.ds(base, ROWS_PT)])


@functools.partial(
    pl.kernel,
    out_type=jax.ShapeDtypeStruct((NW, CPT, CH), jnp.float32),
    mesh=_mesh,
    compiler_params=_sc_params,
    scratch_types=[
        pltpu.VMEM((CPT, CH), jnp.int32),
        pltpu.VMEM((CPT, CH), jnp.int32),
        pltpu.VMEM((CPT, CH), jnp.float32),
        pltpu.VMEM((CPT, CH), jnp.float32),
        pltpu.VMEM((NP,), jnp.float32),
        pltpu.VMEM((NP,), jnp.float32),
    ],
)
def _normw_kernel(src_h, dst_h, ew_h, swdeg_h, dwdeg_h, out_h,
                  srcv, dstv, ewv, outv, swv, dwv):
  cid = lax.axis_index("c")
  sid = lax.axis_index("s")
  wid = _wid(cid, sid)

  pltpu.sync_copy(src_h.at[wid], srcv)
  pltpu.sync_copy(dst_h.at[wid], dstv)
  pltpu.sync_copy(ew_h.at[wid], ewv)
  pltpu.sync_copy(swdeg_h, swv)
  pltpu.sync_copy(dwdeg_h, dwv)

  def chunk(j, carry):
    for g in range(CH // 16):
      s16 = srcv[j, pl.ds(g * 16, 16)]
      d16 = dstv[j, pl.ds(g * 16, 16)]
      s = plsc.load_gather(swv, [s16])
      t = plsc.load_gather(dwv, [d16])
      p = jnp.maximum(s * t, jnp.float32(1e-12))
      xi = lax.bitcast_convert_type(p, jnp.int32)
      yi = jnp.int32(0x5F3759DF) - lax.shift_right_arithmetic(xi, 1)
      y = lax.bitcast_convert_type(yi, jnp.float32)
      for _ in range(3):
        y = y * (jnp.float32(1.5) - jnp.float32(0.5) * p * y * y)
      w16 = ewv[j, pl.ds(g * 16, 16)]
      outv[j, pl.ds(g * 16, 16)] = w16 * y
    return carry

  lax.fori_loop(0, CPT, chunk, 0)
  pltpu.sync_copy(outv, out_h.at[wid])


def _make_edge_pass(weighted):
  scratch = [
      pltpu.VMEM((CPT, CH), jnp.int32),
      pltpu.VMEM((CPT, CH), jnp.int32),
  ]
  if weighted:
    scratch.append(pltpu.VMEM((CPT, CH), jnp.float32))
  scratch += [
      pltpu.VMEM((CH, D), jnp.float32),
      pltpu.VMEM((16, D), jnp.float32),
      pltpu.SemaphoreType.DMA,
      pltpu.VMEM_SHARED((NP, D), jnp.float32),
  ]

  @functools.partial(
      pl.kernel,
      out_type=jax.ShapeDtypeStruct((NC, NP, D), jnp.float32),
      mesh=_mesh,
      compiler_params=_sc_params,
      scratch_types=scratch,
  )
  def edge_pass(h_h, src_h, dst_h, *rest):
    if weighted:
      (w_h, out_h, srcv, dstv, wv, rows, zbuf, sem, acc) = rest
    else:
      (out_h, srcv, dstv, rows, zbuf, sem, acc) = rest
    cid = lax.axis_index("c")
    sid = lax.axis_index("s")
    wid = _wid(cid, sid)

    def zrow(r, carry):
      for g in range(D // 16):
        zbuf[r, pl.ds(g * 16, 16)] = jnp.zeros((16,), jnp.float32)
      return carry

    lax.fori_loop(0, 16, zrow, 0)

    def zacc(i, carry):
      pltpu.sync_copy(zbuf, acc.at[pl.ds(sid * ROWS_PT + i * 16, 16)])
      return carry

    lax.fori_loop(0, ROWS_PT // 16, zacc, 0)

    pltpu.sync_copy(src_h.at[wid], srcv)
    pltpu.sync_copy(dst_h.at[wid], dstv)
    if weighted:
      pltpu.sync_copy(w_h.at[wid], wv)
    plsc.subcore_barrier()

    def chunk(j, carry):
      pltpu.async_copy(h_h.at[srcv.at[j]], rows, sem).wait()
      if weighted:
        eidx = [lax.iota(jnp.int32, 16) + g * 16 for g in range(CH // 16)]
        w16s = [wv[j, pl.ds(g * 16, 16)] for g in range(CH // 16)]

        def scale(f, c2):
          fs = jnp.full((16,), f, jnp.int32)
          for g in range(CH // 16):
            v = plsc.load_gather(rows, [eidx[g], fs])
            plsc.store_scatter(rows, [eidx[g], fs], v * w16s[g])
          return c2

        lax.fori_loop(0, D, scale, 0)
      pltpu.sync_copy(rows, acc.at[dstv.at[j]], add=True)
      return carry

    lax.fori_loop(0, CPT, chunk, 0)
    plsc.subcore_barrier()
    pltpu.sync_copy(acc.at[pl.ds(sid * ROWS_PT, ROWS_PT)],
                    out_h.at[cid, pl.ds(sid * ROWS_PT, ROWS_PT)])

  return edge_pass


_edge_pass_w = _make_edge_pass(True)
_edge_pass_u = _make_edge_pass(False)


_RB = 256


def _node_scalars(deg):
  def body(deg_ref, out_ref):
    d = deg_ref[0] + deg_ref[1]
    out_ref[0:2] = d[0:2]
    out_ref[2:3] = lax.rsqrt(jnp.maximum(d[2:3], jnp.float32(1.0)))
    out_ref[3:4] = lax.rsqrt(jnp.maximum(d[3:4], jnp.float32(1.0)))

  return pl.pallas_call(
      body, out_shape=jax.ShapeDtypeStruct((4, NP), jnp.float32))(deg)


def _row_spec():
  return pl.BlockSpec((_RB, D), lambda i: (i, 0))


def _full_spec(shape):
  return pl.BlockSpec(shape, lambda i: tuple(0 for _ in shape))


def _col_spec():
  return pl.BlockSpec((_RB, 1), lambda i: (i, 0))


def _mm_masked(feats, mask, w):
  def body(f_ref, m_ref, w_ref, o_ref):
    o_ref[...] = jnp.dot(f_ref[...] * m_ref[...], w_ref[...],
                         preferred_element_type=jnp.float32)

  return pl.pallas_call(
      body, grid=(NP // _RB,),
      in_specs=[_row_spec(), _row_spec(), _full_spec((D, D))],
      out_specs=_row_spec(),
      out_shape=jax.ShapeDtypeStruct((NP, D), jnp.float32))(feats, mask, w)


def _mm_layer2(a, b, bias, nsrc, w):
  def body(a_ref, b_ref, bias_ref, n_ref, w_ref, o_ref):
    h = jax.nn.relu(a_ref[...] + b_ref[...] + bias_ref[...])
    o_ref[...] = jnp.dot(h * n_ref[...], w_ref[...],
                         preferred_element_type=jnp.float32)

  return pl.pallas_call(
      body, grid=(NP // _RB,),
      in_specs=[_row_spec(), _row_spec(), _full_spec((1, D)), _col_spec(),
                _full_spec((D, D))],
      out_specs=_row_spec(),
      out_shape=jax.ShapeDtypeStruct((NP, D), jnp.float32))(
          a, b, bias, nsrc, w)


def _mm_layer3(a, b, ndst, bias, nsrc, w):
  def body(a_ref, b_ref, nd_ref, bias_ref, ns_ref, w_ref, o_ref):
    h = jax.nn.relu((a_ref[...] + b_ref[...]) * nd_ref[...] + bias_ref[...])
    o_ref[...] = jnp.dot(h * ns_ref[...], w_ref[...],
                         preferred_element_type=jnp.float32)

  return pl.pallas_call(
      body, grid=(NP // _RB,),
      in_specs=[_row_spec(), _row_spec(), _col_spec(), _full_spec((1, D)),
                _col_spec(), _full_spec((D, D))],
      out_specs=_row_spec(),
      out_shape=jax.ShapeDtypeStruct((NP, D), jnp.float32))(
          a, b, ndst, bias, nsrc, w)


def _mm_out(a, b, ndst, bias):
  def body(a_ref, b_ref, nd_ref, bias_ref, o_ref):
    o_ref[...] = jax.nn.sigmoid(
        (a_ref[...] + b_ref[...]) * nd_ref[...] + bias_ref[...])

  return pl.pallas_call(
      body, grid=(NP // _RB,),
      in_specs=[_row_spec(), _row_spec(), _col_spec(), _full_spec((1, D))],
      out_specs=_row_spec(),
      out_shape=jax.ShapeDtypeStruct((NP, D), jnp.float32))(a, b, ndst, bias)


def kernel(features, edge_index, edge_weight, mask, W1, b1, W2, b2, W3, b3):
  src = edge_index[0]
  dst = edge_index[1]
  pad = EP - NE
  src_p = jnp.concatenate(
      [src, jnp.zeros((pad,), jnp.int32)]).reshape(NW, CPT, CH)
  dst_p = jnp.concatenate(
      [dst, jnp.full((pad,), DUMMY, jnp.int32)]).reshape(NW, CPT, CH)
  ew_p = jnp.concatenate(
      [edge_weight, jnp.zeros((pad,), jnp.float32)]).reshape(NW, CPT, CH)
  one_p = jnp.concatenate(
      [jnp.ones((NE,), jnp.float32),
       jnp.zeros((pad,), jnp.float32)]).reshape(NW, CPT, CH)
  featsp = jnp.pad(features, ((0, NP - NNODE), (0, 0)))
  maskp = jnp.pad(mask, ((0, NP - NNODE), (0, 0)))

  deg = _deg_kernel(src_p, dst_p, ew_p, one_p)
  scal = _node_scalars(deg)
  swdeg, dwdeg = scal[0], scal[1]
  nsrc_col = scal[2][:, None]
  ndst_col = scal[3][:, None]
  normw = _normw_kernel(src_p, dst_p, ew_p, swdeg, dwdeg)

  h1 = _mm_masked(featsp, maskp, W1)
  agg1 = _edge_pass_w(h1, src_p, dst_p, normw)
  h2in = _mm_layer2(agg1[0], agg1[1], b1[None, :], nsrc_col, W2)
  agg2 = _edge_pass_u(h2in, src_p, dst_p)
  h3in = _mm_layer3(agg2[0], agg2[1], ndst_col, b2[None, :], nsrc_col, W3)
  agg3 = _edge_pass_u(h3in, src_p, dst_p)
  out = _mm_out(agg3[0], agg3[1], ndst_col, b3[None, :])
  return out[:NNODE]
